# baseline (speedup 1.0000x reference)
import functools

import jax
import jax.numpy as jnp
from jax import lax
from jax.experimental import pallas as pl
from jax.experimental.pallas import tpu as pltpu
from jax.experimental.pallas import tpu_sc as plsc

NUM_TABLE_ROWS = 1000000
DIM = 32
B_SAMPLES = 16384
SLOTS = 50
NC, NS = 2, 16
NW = NC * NS
NTJ = B_SAMPLES // 128
TJ_PER_W = NTJ // NW
UNITS = SLOTS * TJ_PER_W

_mesh = plsc.VectorSubcoreMesh(core_axis_name="c", subcore_axis_name="s")


@functools.partial(
    pl.kernel,
    mesh=_mesh,
    out_type=jax.ShapeDtypeStruct((SLOTS, 4 * NTJ, 8 * 128), jnp.float32),
    scratch_types=[
        pltpu.VMEM((SLOTS, 128 * TJ_PER_W), jnp.int32),
        [pltpu.VMEM((128, DIM), jnp.float32) for _ in range(4)],
        [pltpu.VMEM((4, 1024), jnp.float32) for _ in range(2)],
        [pltpu.SemaphoreType.DMA for _ in range(4)],
        [pltpu.SemaphoreType.DMA for _ in range(2)],
    ],
    compiler_params=pltpu.CompilerParams(
        use_tc_tiling_on_sc=False, needs_layout_passes=False
    ),
)
def _lookup(idx_hbm, table_hbm, out_hbm, idx_v, rows, tbufs, sem_in, sem_out):
    wid = lax.axis_index("s") * NC + lax.axis_index("c")
    tj0 = wid * TJ_PER_W
    iota = lax.iota(jnp.int32, 16)

    pltpu.sync_copy(idx_hbm.at[:, pl.ds(tj0 * 128, 128 * TJ_PER_W)], idx_v)

    def in_copy(u, b):
        s = jnp.right_shift(u, 2)
        j = jnp.bitwise_and(u, 3)
        return pltpu.make_async_copy(
            table_hbm.at[idx_v.at[s, pl.ds(j * 128, 128)]], rows[b], sem_in[b]
        )

    def out_copy(u, tb, ti):
        s = jnp.right_shift(u, 2)
        tj = tj0 + jnp.bitwise_and(u, 3)
        return pltpu.make_async_copy(
            tbufs[tb].at[ti], out_hbm.at[s, ti * 128 + tj], sem_out[tb]
        )

    def drain_outs(b):
        pltpu.make_async_copy(
            out_hbm.at[0, pl.ds(0, 4)], tbufs[b], sem_out[b]
        ).wait()

    def transpose(b, tb):
        @plsc.parallel_loop(0, 256, unroll=8)
        def _t(v):
            p0 = v * 16
            row_vec = iota + jnp.bitwise_and(p0, 127)
            col_vec = jnp.zeros((16,), jnp.int32) + jnp.right_shift(v, 3)
            ti = jnp.right_shift(v, 6)
            off = pl.multiple_of(jnp.bitwise_and(p0, 1023), 16)
            tbufs[tb][ti, pl.ds(off, 16)] = plsc.load_gather(
                rows[b], [row_vec, col_vec]
            )

    def step(u, b, tb, drain, start_next):
        in_copy(u, b).wait()
        if start_next:
            in_copy(u + 3, (b + 3) % 4).start()
        if drain:
            drain_outs(tb)
        transpose(b, tb)
        for ti in range(4):
            out_copy(u, tb, ti).start()

    for b in range(3):
        in_copy(b, b).start()
    for u0 in range(4):
        step(u0, u0 % 4, u0 % 2, drain=u0 >= 2, start_next=True)

    @pl.loop(1, UNITS // 4 - 1)
    def _grp(g):
        for b in range(4):
            u = 4 * g + b
            step(u, b, b % 2, drain=True, start_next=True)

    for u0 in range(UNITS - 4, UNITS):
        step(u0, u0 % 4, u0 % 2, drain=True, start_next=u0 + 3 < UNITS)
    for tb in range(2):
        drain_outs(tb)


def kernel(indices, table):
    out3d = _lookup(indices.T.astype(jnp.int32), table)
    o = out3d.reshape(SLOTS, 4, 128, 8, 128)
    o = o.transpose(0, 1, 3, 2, 4)
    o = o.reshape(SLOTS, DIM, B_SAMPLES)
    return o.transpose(2, 0, 1)


# speedup vs baseline: 1.1537x; 1.1537x over previous
import functools

import jax
import jax.numpy as jnp
from jax import lax
from jax.experimental import pallas as pl
from jax.experimental.pallas import tpu as pltpu
from jax.experimental.pallas import tpu_sc as plsc

NUM_TABLE_ROWS = 1000000
DIM = 32
B_SAMPLES = 16384
SLOTS = 50
NC, NS = 2, 16
NW = NC * NS
NTJ = B_SAMPLES // 128
TJ_PER_W = NTJ // NW
UNITS = SLOTS * TJ_PER_W

_mesh = plsc.VectorSubcoreMesh(core_axis_name="c", subcore_axis_name="s")


@functools.partial(
    pl.kernel,
    mesh=_mesh,
    out_type=jax.ShapeDtypeStruct((SLOTS, 4 * NTJ, 8 * 128), jnp.float32),
    scratch_types=[
        pltpu.VMEM((SLOTS, 128 * TJ_PER_W), jnp.int32),
        [pltpu.VMEM((128, DIM), jnp.float32) for _ in range(4)],
        [pltpu.VMEM((4, 1024), jnp.float32) for _ in range(2)],
        [pltpu.SemaphoreType.DMA for _ in range(4)],
        [pltpu.SemaphoreType.DMA for _ in range(2)],
    ],
    compiler_params=pltpu.CompilerParams(
        use_tc_tiling_on_sc=False, needs_layout_passes=False
    ),
)
def _lookup(idx_hbm, table_hbm, out_hbm, idx_v, rows, tbufs, sem_in, sem_out):
    wid = lax.axis_index("s") * NC + lax.axis_index("c")
    tj0 = wid * TJ_PER_W
    iota = lax.iota(jnp.int32, 16)
    rowvecs = [iota + 16 * k for k in range(8)]

    pltpu.sync_copy(idx_hbm.at[:, pl.ds(tj0 * 128, 128 * TJ_PER_W)], idx_v)

    def in_copy(u, b):
        s = jnp.right_shift(u, 2)
        j = jnp.bitwise_and(u, 3)
        return pltpu.make_async_copy(
            table_hbm.at[idx_v.at[s, pl.ds(j * 128, 128)]], rows[b], sem_in[b]
        )

    def out_copy(u, tb, ti):
        s = jnp.right_shift(u, 2)
        tj = tj0 + jnp.bitwise_and(u, 3)
        return pltpu.make_async_copy(
            tbufs[tb].at[ti], out_hbm.at[s, ti * 128 + tj], sem_out[tb]
        )

    def drain_outs(b):
        pltpu.make_async_copy(
            out_hbm.at[0, pl.ds(0, 4)], tbufs[b], sem_out[b]
        ).wait()

    def transpose(b, tb):
        @plsc.parallel_loop(0, DIM, unroll=4)
        def _t(d):
            col_vec = jnp.zeros((16,), jnp.int32) + d
            ti = jnp.right_shift(d, 3)
            base = pl.multiple_of(jnp.bitwise_and(d, 7) * 128, 128)
            for k in range(8):
                tbufs[tb][ti, pl.ds(base + 16 * k, 16)] = plsc.load_gather(
                    rows[b], [rowvecs[k], col_vec]
                )

    def step(u, b, tb, drain, start_next):
        in_copy(u, b).wait()
        if start_next:
            in_copy(u + 3, (b + 3) % 4).start()
        if drain:
            drain_outs(tb)
        transpose(b, tb)
        for ti in range(4):
            out_copy(u, tb, ti).start()

    for b in range(3):
        in_copy(b, b).start()
    for u0 in range(4):
        step(u0, u0 % 4, u0 % 2, drain=u0 >= 2, start_next=True)

    @pl.loop(1, UNITS // 4 - 1)
    def _grp(g):
        for b in range(4):
            u = 4 * g + b
            step(u, b, b % 2, drain=True, start_next=True)

    for u0 in range(UNITS - 4, UNITS):
        step(u0, u0 % 4, u0 % 2, drain=True, start_next=u0 + 3 < UNITS)
    for tb in range(2):
        drain_outs(tb)


def kernel(indices, table):
    out3d = _lookup(indices.T.astype(jnp.int32), table)
    o = out3d.reshape(SLOTS, 4, 128, 8, 128)
    o = o.transpose(0, 1, 3, 2, 4)
    o = o.reshape(SLOTS, DIM, B_SAMPLES)
    return o.transpose(2, 0, 1)


# speedup vs baseline: 1.2785x; 1.1081x over previous
import functools

import jax
import jax.numpy as jnp
from jax import lax
from jax.experimental import pallas as pl
from jax.experimental.pallas import tpu as pltpu
from jax.experimental.pallas import tpu_sc as plsc

NUM_TABLE_ROWS = 1000000
DIM = 32
B_SAMPLES = 16384
SLOTS = 50
NC, NS = 2, 16
NW = NC * NS
NTJ = B_SAMPLES // 128
TJ_PER_W = NTJ // NW
UNITS = SLOTS * TJ_PER_W

_mesh = plsc.VectorSubcoreMesh(core_axis_name="c", subcore_axis_name="s")

NBLK = NUM_TABLE_ROWS // 128 + 1
FULL_TRIPS = (NBLK - 5) // NW


@functools.partial(
    pl.kernel,
    mesh=_mesh,
    out_type=jax.ShapeDtypeStruct((NUM_TABLE_ROWS // 4, 128), jnp.float32),
    scratch_types=[
        [pltpu.VMEM((DIM, 128), jnp.float32) for _ in range(2)],
        [pltpu.VMEM((DIM, 128), jnp.float32) for _ in range(2)],
        [pltpu.SemaphoreType.DMA for _ in range(2)],
        [pltpu.SemaphoreType.DMA for _ in range(2)],
    ],
    compiler_params=pltpu.CompilerParams(needs_layout_passes=False),
)
def _transpose_table(tab_hbm, tail_hbm, out_hbm, ins, outs, sem_in, sem_out):
    wid = lax.axis_index("s") * NC + lax.axis_index("c")
    iota = lax.iota(jnp.int32, 16)
    dvecs = [iota, iota + 16]

    def in_copy(j, b, width=128):
        return pltpu.make_async_copy(
            tab_hbm.at[:, pl.ds(j * 128, width)],
            ins[b].at[:, pl.ds(0, width)],
            sem_in[b],
        )

    def out_copy(j, b, nrows=DIM):
        return pltpu.make_async_copy(
            outs[b].at[pl.ds(0, nrows)],
            out_hbm.at[pl.ds(j * DIM, nrows)],
            sem_out[b],
        )

    def drain_out(b, nrows=DIM):
        pltpu.make_async_copy(
            out_hbm.at[pl.ds(0, nrows)],
            outs[b].at[pl.ds(0, nrows)],
            sem_out[b],
        ).wait()

    def transpose(b, rmax=DIM):
        @plsc.parallel_loop(0, rmax, unroll=2)
        def _t(r):
            r4 = r * 4
            for k in range(8):
                col_vec = jnp.zeros((16,), jnp.int32) + (r4 + k // 2)
                outs[b][r, pl.ds(16 * k, 16)] = plsc.load_gather(
                    ins[b], [dvecs[k % 2], col_vec]
                )

    ntrips = jnp.where(wid < 4, FULL_TRIPS + 1, FULL_TRIPS)

    in_copy(wid, 0).start()

    @pl.loop(0, (FULL_TRIPS + 2) // 2)
    def _pair(g):
        for b in range(2):
            t = 2 * g + b
            j = wid + NW * t

            def _active(j=j, t=t, b=b, g=g):
                in_copy(j, b).wait()
                pl.when(t + 1 < ntrips)(
                    lambda: in_copy(wid + NW * (t + 1), 1 - b).start()
                )
                pl.when(g > 0)(lambda: drain_out(b))
                transpose(b)
                out_copy(j, b).start()

            pl.when(t < ntrips)(_active)

    for b in range(2):
        drain_out(b)

    @pl.when(wid == 4)
    def _tail():
        pltpu.sync_copy(tail_hbm, outs[0].at[pl.ds(0, 16)])
        out_copy(7812, 0, nrows=16).start()
        drain_out(0, nrows=16)


@functools.partial(
    pl.kernel,
    mesh=_mesh,
    out_type=jax.ShapeDtypeStruct((SLOTS, 4 * NTJ, 8 * 128), jnp.float32),
    scratch_types=[
        pltpu.VMEM((SLOTS, 128 * TJ_PER_W), jnp.int32),
        [pltpu.VMEM((128, DIM), jnp.float32) for _ in range(4)],
        [pltpu.VMEM((4, 1024), jnp.float32) for _ in range(2)],
        [pltpu.SemaphoreType.DMA for _ in range(4)],
        [pltpu.SemaphoreType.DMA for _ in range(2)],
    ],
    compiler_params=pltpu.CompilerParams(
        use_tc_tiling_on_sc=False, needs_layout_passes=False
    ),
)
def _lookup(idx_hbm, table_hbm, out_hbm, idx_v, rows, tbufs, sem_in, sem_out):
    wid = lax.axis_index("s") * NC + lax.axis_index("c")
    tj0 = wid * TJ_PER_W
    iota = lax.iota(jnp.int32, 16)
    rowvecs = [iota + 16 * k for k in range(8)]

    pltpu.sync_copy(idx_hbm.at[:, pl.ds(tj0 * 128, 128 * TJ_PER_W)], idx_v)

    def in_copy(u, b):
        s = jnp.right_shift(u, 2)
        j = jnp.bitwise_and(u, 3)
        return pltpu.make_async_copy(
            table_hbm.at[idx_v.at[s, pl.ds(j * 128, 128)]], rows[b], sem_in[b]
        )

    def out_copy(u, tb, ti):
        s = jnp.right_shift(u, 2)
        tj = tj0 + jnp.bitwise_and(u, 3)
        return pltpu.make_async_copy(
            tbufs[tb].at[ti], out_hbm.at[s, ti * 128 + tj], sem_out[tb]
        )

    def drain_outs(b):
        pltpu.make_async_copy(
            out_hbm.at[0, pl.ds(0, 4)], tbufs[b], sem_out[b]
        ).wait()

    def transpose(b, tb):
        @plsc.parallel_loop(0, DIM, unroll=4)
        def _t(d):
            col_vec = jnp.zeros((16,), jnp.int32) + d
            ti = jnp.right_shift(d, 3)
            base = pl.multiple_of(jnp.bitwise_and(d, 7) * 128, 128)
            for k in range(8):
                tbufs[tb][ti, pl.ds(base + 16 * k, 16)] = plsc.load_gather(
                    rows[b], [rowvecs[k], col_vec]
                )

    def step(u, b, tb, drain, start_next):
        in_copy(u, b).wait()
        if start_next:
            in_copy(u + 3, (b + 3) % 4).start()
        if drain:
            drain_outs(tb)
        transpose(b, tb)
        for ti in range(4):
            out_copy(u, tb, ti).start()

    for b in range(3):
        in_copy(b, b).start()
    for u0 in range(4):
        step(u0, u0 % 4, u0 % 2, drain=u0 >= 2, start_next=True)

    @pl.loop(1, UNITS // 4 - 1)
    def _grp(g):
        for b in range(4):
            u = 4 * g + b
            step(u, b, b % 2, drain=True, start_next=True)

    for u0 in range(UNITS - 4, UNITS):
        step(u0, u0 % 4, u0 % 2, drain=True, start_next=u0 + 3 < UNITS)
    for tb in range(2):
        drain_outs(tb)


def kernel(indices, table):
    tail2d = table[NUM_TABLE_ROWS - 64 :].reshape(16, 128)
    tab_rm = _transpose_table(table.T, tail2d).reshape(NUM_TABLE_ROWS, DIM)
    out3d = _lookup(indices.T.astype(jnp.int32), tab_rm)
    o = out3d.reshape(SLOTS, 4, 128, 8, 128)
    o = o.transpose(0, 1, 3, 2, 4)
    o = o.reshape(SLOTS, DIM, B_SAMPLES)
    return o.transpose(2, 0, 1)
